# baseline (speedup 1.0000x reference)
import functools

import jax
import jax.numpy as jnp
from jax import lax
from jax.experimental import pallas as pl
from jax.experimental.pallas import tpu as pltpu
from jax.experimental.pallas import tpu_sc as plsc

_N = 10000
_E = 320000
_D = 128
_H = 256
_EPS = 1e-5

_NW = 32
_ROWS = 320
_NPAD = _NW * _ROWS
_C = 3200
_NCHUNK = _E // _C
_NPAIR = _NCHUNK // 2
_G = _C // 16
_ACC = _ROWS * 8
_ACCP = _ACC + 8

_mesh = plsc.VectorSubcoreMesh(core_axis_name="c", subcore_axis_name="s")


@functools.partial(
    pl.kernel,
    out_type=[
        jax.ShapeDtypeStruct((_NPAD * 8, 16), jnp.float32),
        jax.ShapeDtypeStruct((_NPAD * 8, 16), jnp.float32),
    ],
    mesh=_mesh,
    compiler_params=pltpu.CompilerParams(
        needs_layout_passes=False, use_tc_tiling_on_sc=False),
    scratch_types=[
        pltpu.VMEM((2, _C), jnp.int32),
        pltpu.VMEM((2, _C), jnp.int32),
        pltpu.VMEM((_C + 64,), jnp.int32),
        pltpu.VMEM((_C + 64,), jnp.int32),
        pltpu.VMEM((2, 64, _D), jnp.float32),
        pltpu.VMEM((_ACCP, 16), jnp.float32),
        pltpu.VMEM((_ACCP, 16), jnp.float32),
        pltpu.SemaphoreType.DMA,
        pltpu.SemaphoreType.DMA,
        pltpu.SemaphoreType.DMA,
        pltpu.SemaphoreType.DMA,
    ],
)
def _sc_agg(src_hbm, dst_hbm, x_hbm, sum_hbm, max_hbm,
            src_v, dst_v, msrc, mld8, rows_v, acc_s, acc_m,
            sem_a, sem_b, sem_r0, sem_r1):
    cid = lax.axis_index("c")
    sid = lax.axis_index("s")
    wid = sid * 2 + cid
    lo = wid * _ROWS

    zero16f = jnp.zeros((16,), jnp.float32)
    ninf16 = jnp.full((16,), -jnp.inf, jnp.float32)
    zero16i = jnp.zeros((16,), jnp.int32)
    pad16i = jnp.full((16,), _ACC, jnp.int32)
    iota16 = lax.iota(jnp.int32, 16)
    full_m = iota16 >= 0

    def init_body(i, carry):
        acc_s[i, :] = zero16f
        acc_m[i, :] = ninf16
        return carry

    lax.fori_loop(0, _ACCP, init_body, 0, unroll=8)

    def start_chunk(c, buf):
        pltpu.async_copy(src_hbm.at[pl.ds(c * _C, _C)], src_v.at[buf], sem_a if buf == 0 else sem_b)
        pltpu.async_copy(dst_hbm.at[pl.ds(c * _C, _C)], dst_v.at[buf], sem_a if buf == 0 else sem_b)

    def wait_chunk(buf):
        sem = sem_a if buf == 0 else sem_b
        pltpu.make_async_copy(src_hbm.at[pl.ds(0, _C)], src_v.at[buf], sem).wait()
        pltpu.make_async_copy(dst_hbm.at[pl.ds(0, _C)], dst_v.at[buf], sem).wait()

    def start_gather(g, rbuf):
        sem = sem_r0 if rbuf == 0 else sem_r1
        for k in range(4):
            b = pl.multiple_of(g * 64 + k * 16, 16)
            sv = msrc[pl.ds(b, 16)]
            pltpu.async_copy(x_hbm.at[sv], rows_v.at[rbuf, pl.ds(k * 16, 16)], sem)

    def wait_gather(rbuf):
        sem = sem_r0 if rbuf == 0 else sem_r1
        for k in range(4):
            pltpu.make_async_copy(
                x_hbm.at[pl.ds(0, 16)], rows_v.at[rbuf, pl.ds(k * 16, 16)], sem).wait()

    def process_group(g, rbuf):
        for sub in range(4):
            b = pl.multiple_of(g * 64 + sub * 16, 16)
            ldv = mld8[pl.ds(b, 16)]
            rbase = sub * 16
            for r in range(16):
                ldx = ldv[r]
                for f in range(8):
                    row = rows_v[rbuf, rbase + r, pl.ds(f * 16, 16)]
                    plsc.addupdate(acc_s.at[ldx + f, :], row)
                    acc_m[ldx + f, :] = jnp.maximum(acc_m[ldx + f, :], row)

    def do_chunk(buf):
        def scan_body(g, cnt):
            b = pl.multiple_of(g * 16, 16)
            d = dst_v[buf, pl.ds(b, 16)]
            s = src_v[buf, pl.ds(b, 16)]
            m = (d >= lo) & (d < lo + _ROWS)
            ld8 = (d - lo) * 8
            plsc.store_compressed(msrc.at[pl.ds(cnt, 16)], s, mask=m)
            plsc.store_compressed(mld8.at[pl.ds(cnt, 16)], ld8, mask=m)
            return cnt + plsc.all_reduce_population_count(m)[0]

        r_tot = lax.fori_loop(0, _G, scan_body, 0, unroll=4)
        for k in range(4):
            plsc.store_scatter(msrc, [r_tot + iota16 + 16 * k], zero16i, mask=full_m)
            plsc.store_scatter(mld8, [r_tot + iota16 + 16 * k], pad16i, mask=full_m)
        ng = (r_tot + 63) >> 6
        ngp = (ng + 1) >> 1

        @pl.when(ng > 0)
        def _():
            start_gather(0, 0)

        def pair_body(p, carry):
            g0 = p * 2
            g1 = g0 + 1

            @pl.when(g1 < ng)
            def _():
                start_gather(g1, 1)

            wait_gather(0)
            process_group(g0, 0)

            @pl.when(g0 + 2 < ng)
            def _():
                start_gather(g0 + 2, 0)

            @pl.when(g1 < ng)
            def _():
                wait_gather(1)
                process_group(g1, 1)

            return carry

        lax.fori_loop(0, ngp, pair_body, 0)

    start_chunk(0, 0)

    def chunk_pair(i, carry):
        start_chunk(i * 2 + 1, 1)
        wait_chunk(0)
        do_chunk(0)

        @pl.when(i + 1 < _NPAIR)
        def _():
            start_chunk(i * 2 + 2, 0)

        wait_chunk(1)
        do_chunk(1)
        return carry

    lax.fori_loop(0, _NPAIR, chunk_pair, 0)

    def fix_body(i, carry):
        v = acc_m[i, :]
        acc_m[i, :] = jnp.where(v == ninf16, zero16f, v)
        return carry

    lax.fori_loop(0, _ACC, fix_body, 0, unroll=8)

    obase = wid * _ACC
    pltpu.sync_copy(acc_s.at[pl.ds(0, _ACC), :], sum_hbm.at[pl.ds(obase, _ACC), :])
    pltpu.sync_copy(acc_m.at[pl.ds(0, _ACC), :], max_hbm.at[pl.ds(obase, _ACC), :])


def _tc_body(x_ref, s_ref, mx_ref, Wm_ref, bm_ref, Wih_ref, Whh_ref,
             bih_ref, bhh_ref, W1_ref, b1_ref, g1_ref, be1_ref,
             W2_ref, b2_ref, g2_ref, be2_ref, o_ref):
    x = x_ref[:]
    Wm = Wm_ref[:]
    merged = (
        jnp.dot(s_ref[:], Wm[:_D], preferred_element_type=jnp.float32)
        + jnp.dot(mx_ref[:], Wm[_D:], preferred_element_type=jnp.float32)
        + bm_ref[:]
    )
    gi = jnp.dot(merged, Wih_ref[:], preferred_element_type=jnp.float32) + bih_ref[:]
    gh = jnp.dot(x, Whh_ref[:], preferred_element_type=jnp.float32) + bhh_ref[:]
    r = jax.nn.sigmoid(gi[:, :_D] + gh[:, :_D])
    z = jax.nn.sigmoid(gi[:, _D:2 * _D] + gh[:, _D:2 * _D])
    n = jnp.tanh(gi[:, 2 * _D:] + r * gh[:, 2 * _D:])
    h = (1.0 - z) * n + z * x

    t1 = jnp.dot(h, W1_ref[:], preferred_element_type=jnp.float32) + b1_ref[:]
    t1 = jnp.where(t1 >= 0, t1, 0.01 * t1)
    m1 = jnp.mean(t1, axis=0)
    v1 = jnp.mean((t1 - m1) * (t1 - m1), axis=0)
    t1 = g1_ref[:] * (t1 - m1) * lax.rsqrt(v1 + _EPS) + be1_ref[:]

    t2 = jnp.dot(t1, W2_ref[:], preferred_element_type=jnp.float32) + b2_ref[:]
    t2 = jnp.where(t2 >= 0, t2, 0.01 * t2)
    m2 = jnp.mean(t2, axis=0)
    v2 = jnp.mean((t2 - m2) * (t2 - m2), axis=0)
    o_ref[:] = g2_ref[:] * (t2 - m2) * lax.rsqrt(v2 + _EPS) + be2_ref[:]


def kernel(x, edges, W_merge, b_merge, W_ih, W_hh, b_ih, b_hh,
           W1, b1, g1, be1, W2, b2, g2, be2):
    src = edges[0]
    dst = edges[1]
    s16, m16 = _sc_agg(src, dst, x)
    agg_sum = s16.reshape(_NPAD, _D)[:_N]
    agg_max = m16.reshape(_NPAD, _D)[:_N]
    out = pl.pallas_call(
        _tc_body,
        out_shape=jax.ShapeDtypeStruct((_N, _D), jnp.float32),
    )(x, agg_sum, agg_max, W_merge, b_merge, W_ih, W_hh, b_ih, b_hh,
      W1, b1, g1, be1, W2, b2, g2, be2)
    return out


# speedup vs baseline: 2.8357x; 2.8357x over previous
import functools

import jax
import jax.numpy as jnp
from jax import lax
from jax.experimental import pallas as pl
from jax.experimental.pallas import tpu as pltpu
from jax.experimental.pallas import tpu_sc as plsc

_N = 10000
_E = 320000
_D = 128
_H = 256
_EPS = 1e-5

_NW = 32
_ROWS = 320
_NPAD = _NW * _ROWS
_C = 3200
_NCHUNK = _E // _C
_NPAIR = _NCHUNK // 2
_G = _C // 16
_ACC = _ROWS * 8
_ACCP = _ACC + 8

_mesh = plsc.VectorSubcoreMesh(core_axis_name="c", subcore_axis_name="s")


@functools.partial(
    pl.kernel,
    out_type=[
        jax.ShapeDtypeStruct((_NPAD * 8, 16), jnp.float32),
        jax.ShapeDtypeStruct((_NPAD * 8, 16), jnp.float32),
    ],
    mesh=_mesh,
    compiler_params=pltpu.CompilerParams(
        needs_layout_passes=False, use_tc_tiling_on_sc=False),
    scratch_types=[
        pltpu.VMEM((2, _C), jnp.int32),
        pltpu.VMEM((2, _C), jnp.int32),
        pltpu.VMEM((_C + 16,), jnp.int32),
        pltpu.VMEM((_C + 16,), jnp.int32),
        pltpu.VMEM((4, 16, _D), jnp.float32),
        pltpu.VMEM((_ACCP, 16), jnp.float32),
        pltpu.VMEM((_ACCP, 16), jnp.float32),
        pltpu.SemaphoreType.DMA,
        pltpu.SemaphoreType.DMA,
        pltpu.SemaphoreType.DMA,
        pltpu.SemaphoreType.DMA,
        pltpu.SemaphoreType.DMA,
        pltpu.SemaphoreType.DMA,
    ],
)
def _sc_agg(src_hbm, dst_hbm, x_hbm, sum_hbm, max_hbm,
            src_v, dst_v, msrc, mld8, rows_v, acc_s, acc_m,
            sem_a, sem_b, sem_r0, sem_r1, sem_r2, sem_r3):
    cid = lax.axis_index("c")
    sid = lax.axis_index("s")
    wid = sid * 2 + cid
    lo = wid * _ROWS

    zero16f = jnp.zeros((16,), jnp.float32)
    ninf16 = jnp.full((16,), -jnp.inf, jnp.float32)
    zero16i = jnp.zeros((16,), jnp.int32)
    pad16i = jnp.full((16,), _ACC, jnp.int32)
    iota16 = lax.iota(jnp.int32, 16)
    full_m = iota16 >= 0

    def init_body(i, carry):
        acc_s[i, :] = zero16f
        acc_m[i, :] = ninf16
        return carry

    lax.fori_loop(0, _ACCP, init_body, 0, unroll=8)

    def start_chunk(c, buf):
        pltpu.async_copy(src_hbm.at[pl.ds(c * _C, _C)], src_v.at[buf], sem_a if buf == 0 else sem_b)
        pltpu.async_copy(dst_hbm.at[pl.ds(c * _C, _C)], dst_v.at[buf], sem_a if buf == 0 else sem_b)

    def wait_chunk(buf):
        sem = sem_a if buf == 0 else sem_b
        pltpu.make_async_copy(src_hbm.at[pl.ds(0, _C)], src_v.at[buf], sem).wait()
        pltpu.make_async_copy(dst_hbm.at[pl.ds(0, _C)], dst_v.at[buf], sem).wait()

    _rsems = (sem_r0, sem_r1, sem_r2, sem_r3)

    def start_gather(g, rbuf):
        b = pl.multiple_of(g * 16, 16)
        sv = msrc[pl.ds(b, 16)]
        pltpu.async_copy(x_hbm.at[sv], rows_v.at[rbuf], _rsems[rbuf])

    def wait_gather(rbuf):
        pltpu.make_async_copy(x_hbm.at[pl.ds(0, 16)], rows_v.at[rbuf], _rsems[rbuf]).wait()

    def process_group(g, rbuf):
        b = pl.multiple_of(g * 16, 16)
        ldv = mld8[pl.ds(b, 16)]
        for r in range(16):
            ldx = ldv[r]
            for f in range(8):
                row = rows_v[rbuf, r, pl.ds(f * 16, 16)]
                plsc.addupdate(acc_s.at[ldx + f, :], row)
                acc_m[ldx + f, :] = jnp.maximum(acc_m[ldx + f, :], row)

    def do_chunk(buf):
        def scan_body(g, cnt):
            b = pl.multiple_of(g * 16, 16)
            d = dst_v[buf, pl.ds(b, 16)]
            s = src_v[buf, pl.ds(b, 16)]
            m = (d >= lo) & (d < lo + _ROWS)
            ld8 = (d - lo) * 8
            plsc.store_compressed(msrc.at[pl.ds(cnt, 16)], s, mask=m)
            plsc.store_compressed(mld8.at[pl.ds(cnt, 16)], ld8, mask=m)
            return cnt + plsc.all_reduce_population_count(m)[0]

        r_tot = lax.fori_loop(0, _G, scan_body, 0, unroll=4)
        plsc.store_scatter(msrc, [r_tot + iota16], zero16i, mask=full_m)
        plsc.store_scatter(mld8, [r_tot + iota16], pad16i, mask=full_m)
        ng = (r_tot + 15) >> 4
        ngq = (ng + 3) >> 2

        for k in range(4):
            @pl.when(k < ng)
            def _():
                start_gather(k, k)

        def quad_body(q, carry):
            g0 = q * 4
            for k in range(4):
                g = g0 + k

                @pl.when(g < ng)
                def _():
                    wait_gather(k)
                    process_group(g, k)

                @pl.when(g + 4 < ng)
                def _():
                    start_gather(g + 4, k)

            return carry

        lax.fori_loop(0, ngq, quad_body, 0)

    start_chunk(0, 0)

    def chunk_pair(i, carry):
        start_chunk(i * 2 + 1, 1)
        wait_chunk(0)
        do_chunk(0)

        @pl.when(i + 1 < _NPAIR)
        def _():
            start_chunk(i * 2 + 2, 0)

        wait_chunk(1)
        do_chunk(1)
        return carry

    lax.fori_loop(0, _NPAIR, chunk_pair, 0)

    def fix_body(i, carry):
        v = acc_m[i, :]
        acc_m[i, :] = jnp.where(v == ninf16, zero16f, v)
        return carry

    lax.fori_loop(0, _ACC, fix_body, 0, unroll=8)

    obase = wid * _ACC
    pltpu.sync_copy(acc_s.at[pl.ds(0, _ACC), :], sum_hbm.at[pl.ds(obase, _ACC), :])
    pltpu.sync_copy(acc_m.at[pl.ds(0, _ACC), :], max_hbm.at[pl.ds(obase, _ACC), :])


def _tc_body(x_ref, s_ref, mx_ref, Wm_ref, bm_ref, Wih_ref, Whh_ref,
             bih_ref, bhh_ref, W1_ref, b1_ref, g1_ref, be1_ref,
             W2_ref, b2_ref, g2_ref, be2_ref, o_ref):
    x = x_ref[:]
    Wm = Wm_ref[:]
    merged = (
        jnp.dot(s_ref[:], Wm[:_D], preferred_element_type=jnp.float32)
        + jnp.dot(mx_ref[:], Wm[_D:], preferred_element_type=jnp.float32)
        + bm_ref[:]
    )
    gi = jnp.dot(merged, Wih_ref[:], preferred_element_type=jnp.float32) + bih_ref[:]
    gh = jnp.dot(x, Whh_ref[:], preferred_element_type=jnp.float32) + bhh_ref[:]
    r = jax.nn.sigmoid(gi[:, :_D] + gh[:, :_D])
    z = jax.nn.sigmoid(gi[:, _D:2 * _D] + gh[:, _D:2 * _D])
    n = jnp.tanh(gi[:, 2 * _D:] + r * gh[:, 2 * _D:])
    h = (1.0 - z) * n + z * x

    t1 = jnp.dot(h, W1_ref[:], preferred_element_type=jnp.float32) + b1_ref[:]
    t1 = jnp.where(t1 >= 0, t1, 0.01 * t1)
    m1 = jnp.mean(t1, axis=0)
    v1 = jnp.mean((t1 - m1) * (t1 - m1), axis=0)
    t1 = g1_ref[:] * (t1 - m1) * lax.rsqrt(v1 + _EPS) + be1_ref[:]

    t2 = jnp.dot(t1, W2_ref[:], preferred_element_type=jnp.float32) + b2_ref[:]
    t2 = jnp.where(t2 >= 0, t2, 0.01 * t2)
    m2 = jnp.mean(t2, axis=0)
    v2 = jnp.mean((t2 - m2) * (t2 - m2), axis=0)
    o_ref[:] = g2_ref[:] * (t2 - m2) * lax.rsqrt(v2 + _EPS) + be2_ref[:]


def kernel(x, edges, W_merge, b_merge, W_ih, W_hh, b_ih, b_hh,
           W1, b1, g1, be1, W2, b2, g2, be2):
    src = edges[0]
    dst = edges[1]
    s16, m16 = _sc_agg(src, dst, x)
    agg_sum = s16.reshape(_NPAD, _D)[:_N]
    agg_max = m16.reshape(_NPAD, _D)[:_N]
    out = pl.pallas_call(
        _tc_body,
        out_shape=jax.ShapeDtypeStruct((_N, _D), jnp.float32),
    )(x, agg_sum, agg_max, W_merge, b_merge, W_ih, W_hh, b_ih, b_hh,
      W1, b1, g1, be1, W2, b2, g2, be2)
    return out


# speedup vs baseline: 3.1090x; 1.0964x over previous
import functools

import jax
import jax.numpy as jnp
from jax import lax
from jax.experimental import pallas as pl
from jax.experimental.pallas import tpu as pltpu
from jax.experimental.pallas import tpu_sc as plsc

_N = 10000
_E = 320000
_D = 128
_H = 256
_EPS = 1e-5

_NW = 32
_ROWS = 320
_NPAD = _NW * _ROWS
_C = 3200
_NCHUNK = _E // _C
_NPAIR = _NCHUNK // 2
_G = _C // 16
_ACC = _ROWS * 8
_ACCP = _ACC + 8

_mesh = plsc.VectorSubcoreMesh(core_axis_name="c", subcore_axis_name="s")


@functools.partial(
    pl.kernel,
    out_type=[
        jax.ShapeDtypeStruct((_NPAD * 8, 16), jnp.float32),
        jax.ShapeDtypeStruct((_NPAD * 8, 16), jnp.float32),
    ],
    mesh=_mesh,
    compiler_params=pltpu.CompilerParams(
        needs_layout_passes=False, use_tc_tiling_on_sc=False),
    scratch_types=[
        pltpu.VMEM((2, _C), jnp.int32),
        pltpu.VMEM((2, _C), jnp.int32),
        pltpu.VMEM((_C + 16,), jnp.int32),
        pltpu.VMEM((_C + 16,), jnp.int32),
        pltpu.VMEM((4, 16, _D), jnp.float32),
        pltpu.VMEM((_ACCP, 16), jnp.float32),
        pltpu.VMEM((_ACCP, 16), jnp.float32),
        pltpu.SemaphoreType.DMA,
        pltpu.SemaphoreType.DMA,
        pltpu.SemaphoreType.DMA,
        pltpu.SemaphoreType.DMA,
        pltpu.SemaphoreType.DMA,
        pltpu.SemaphoreType.DMA,
    ],
)
def _sc_agg(src_hbm, dst_hbm, x_hbm, sum_hbm, max_hbm,
            src_v, dst_v, msrc, mld8, rows_v, acc_s, acc_m,
            sem_a, sem_b, sem_r0, sem_r1, sem_r2, sem_r3):
    cid = lax.axis_index("c")
    sid = lax.axis_index("s")
    wid = sid * 2 + cid
    lo = wid * _ROWS

    zero16f = jnp.zeros((16,), jnp.float32)
    ninf16 = jnp.full((16,), -jnp.inf, jnp.float32)
    zero16i = jnp.zeros((16,), jnp.int32)
    pad16i = jnp.full((16,), _ACC, jnp.int32)
    iota16 = lax.iota(jnp.int32, 16)
    full_m = iota16 >= 0

    def init_body(i, carry):
        acc_s[i, :] = zero16f
        acc_m[i, :] = ninf16
        return carry

    lax.fori_loop(0, _ACCP, init_body, 0, unroll=8)

    def start_chunk(c, buf):
        pltpu.async_copy(src_hbm.at[pl.ds(c * _C, _C)], src_v.at[buf], sem_a if buf == 0 else sem_b)
        pltpu.async_copy(dst_hbm.at[pl.ds(c * _C, _C)], dst_v.at[buf], sem_a if buf == 0 else sem_b)

    def wait_chunk(buf):
        sem = sem_a if buf == 0 else sem_b
        pltpu.make_async_copy(src_hbm.at[pl.ds(0, _C)], src_v.at[buf], sem).wait()
        pltpu.make_async_copy(dst_hbm.at[pl.ds(0, _C)], dst_v.at[buf], sem).wait()

    _rsems = (sem_r0, sem_r1, sem_r2, sem_r3)

    def start_gather(g, rbuf):
        b = pl.multiple_of(g * 16, 16)
        sv = msrc[pl.ds(b, 16)]
        pltpu.async_copy(x_hbm.at[sv], rows_v.at[rbuf], _rsems[rbuf])

    def wait_gather(rbuf):
        pltpu.make_async_copy(x_hbm.at[pl.ds(0, 16)], rows_v.at[rbuf], _rsems[rbuf]).wait()

    def process_group(g, rbuf):
        b = pl.multiple_of(g * 16, 16)
        ldv = mld8[pl.ds(b, 16)]
        for r in range(16):
            ldx = ldv[r]
            for f in range(8):
                row = rows_v[rbuf, r, pl.ds(f * 16, 16)]
                plsc.addupdate(acc_s.at[ldx + f, :], row)
                acc_m[ldx + f, :] = jnp.maximum(acc_m[ldx + f, :], row)

    def do_chunk(buf):
        def scan_body(g, cnt):
            b = pl.multiple_of(g * 16, 16)
            d = dst_v[buf, pl.ds(b, 16)]
            s = src_v[buf, pl.ds(b, 16)]
            m = (d >= lo) & (d < lo + _ROWS)
            ld8 = (d - lo) * 8
            plsc.store_compressed(msrc.at[pl.ds(cnt, 16)], s, mask=m)
            plsc.store_compressed(mld8.at[pl.ds(cnt, 16)], ld8, mask=m)
            return cnt + plsc.all_reduce_population_count(m)[0]

        r_tot = lax.fori_loop(0, _G, scan_body, 0, unroll=4)
        plsc.store_scatter(msrc, [r_tot + iota16], iota16, mask=full_m)
        plsc.store_scatter(mld8, [r_tot + iota16], pad16i, mask=full_m)
        ng = (r_tot + 15) >> 4
        ngq = (ng + 3) >> 2

        for k in range(4):
            @pl.when(k < ng)
            def _():
                start_gather(k, k)

        def quad_body(q, carry):
            g0 = q * 4
            for k in range(4):
                g = g0 + k

                @pl.when(g < ng)
                def _():
                    wait_gather(k)
                    process_group(g, k)

                @pl.when(g + 4 < ng)
                def _():
                    start_gather(g + 4, k)

            return carry

        lax.fori_loop(0, ngq, quad_body, 0)

    start_chunk(0, 0)

    def chunk_pair(i, carry):
        start_chunk(i * 2 + 1, 1)
        wait_chunk(0)
        do_chunk(0)

        @pl.when(i + 1 < _NPAIR)
        def _():
            start_chunk(i * 2 + 2, 0)

        wait_chunk(1)
        do_chunk(1)
        return carry

    lax.fori_loop(0, _NPAIR, chunk_pair, 0)

    def fix_body(i, carry):
        v = acc_m[i, :]
        acc_m[i, :] = jnp.where(v == ninf16, zero16f, v)
        return carry

    lax.fori_loop(0, _ACC, fix_body, 0, unroll=8)

    obase = wid * _ACC
    pltpu.sync_copy(acc_s.at[pl.ds(0, _ACC), :], sum_hbm.at[pl.ds(obase, _ACC), :])
    pltpu.sync_copy(acc_m.at[pl.ds(0, _ACC), :], max_hbm.at[pl.ds(obase, _ACC), :])


def _tc_body(x_ref, s_ref, mx_ref, Wm_ref, bm_ref, Wih_ref, Whh_ref,
             bih_ref, bhh_ref, W1_ref, b1_ref, g1_ref, be1_ref,
             W2_ref, b2_ref, g2_ref, be2_ref, o_ref):
    x = x_ref[:]
    Wm = Wm_ref[:]
    merged = (
        jnp.dot(s_ref[:], Wm[:_D], preferred_element_type=jnp.float32)
        + jnp.dot(mx_ref[:], Wm[_D:], preferred_element_type=jnp.float32)
        + bm_ref[:]
    )
    gi = jnp.dot(merged, Wih_ref[:], preferred_element_type=jnp.float32) + bih_ref[:]
    gh = jnp.dot(x, Whh_ref[:], preferred_element_type=jnp.float32) + bhh_ref[:]
    r = jax.nn.sigmoid(gi[:, :_D] + gh[:, :_D])
    z = jax.nn.sigmoid(gi[:, _D:2 * _D] + gh[:, _D:2 * _D])
    n = jnp.tanh(gi[:, 2 * _D:] + r * gh[:, 2 * _D:])
    h = (1.0 - z) * n + z * x

    t1 = jnp.dot(h, W1_ref[:], preferred_element_type=jnp.float32) + b1_ref[:]
    t1 = jnp.where(t1 >= 0, t1, 0.01 * t1)
    m1 = jnp.mean(t1, axis=0)
    v1 = jnp.mean((t1 - m1) * (t1 - m1), axis=0)
    t1 = g1_ref[:] * (t1 - m1) * lax.rsqrt(v1 + _EPS) + be1_ref[:]

    t2 = jnp.dot(t1, W2_ref[:], preferred_element_type=jnp.float32) + b2_ref[:]
    t2 = jnp.where(t2 >= 0, t2, 0.01 * t2)
    m2 = jnp.mean(t2, axis=0)
    v2 = jnp.mean((t2 - m2) * (t2 - m2), axis=0)
    o_ref[:] = g2_ref[:] * (t2 - m2) * lax.rsqrt(v2 + _EPS) + be2_ref[:]


def kernel(x, edges, W_merge, b_merge, W_ih, W_hh, b_ih, b_hh,
           W1, b1, g1, be1, W2, b2, g2, be2):
    src = edges[0]
    dst = edges[1]
    s16, m16 = _sc_agg(src, dst, x)
    agg_sum = s16.reshape(_NPAD, _D)[:_N]
    agg_max = m16.reshape(_NPAD, _D)[:_N]
    out = pl.pallas_call(
        _tc_body,
        out_shape=jax.ShapeDtypeStruct((_N, _D), jnp.float32),
    )(x, agg_sum, agg_max, W_merge, b_merge, W_ih, W_hh, b_ih, b_hh,
      W1, b1, g1, be1, W2, b2, g2, be2)
    return out
